# Initial kernel scaffold; baseline (speedup 1.0000x reference)
import functools

import jax
import jax.numpy as jnp
from jax import lax
from jax.experimental import pallas as pl
from jax.experimental.pallas import tpu as pltpu
from jax.experimental.pallas import tpu_sc as plsc

A = 64
M = 4096
C = 64
K = 20
KPAD = 24

NWORK = 32
NC2CHUNK = 4
C2W = C // NC2CHUNK
NIGRP = NWORK // NC2CHUNK
IPG = A // NIGRP


def _softmax_topk_body(mm_ref, sm_ref, ind_ref, v_ref):
    x = mm_ref[0]
    xm = jnp.max(x, axis=0, keepdims=True)
    e = jnp.exp(x - xm)
    s = jnp.sum(e, axis=0, keepdims=True)
    sm = e / s
    sm_ref[0] = sm
    iota_m = lax.broadcasted_iota(jnp.int32, (M, C), 0)
    work = sm
    for t in range(K):
        mx = jnp.max(work, axis=0)
        cand = jnp.where(work == mx[None, :], iota_m, M)
        idx = jnp.min(cand, axis=0)
        v_ref[0, t : t + 1, :] = mx[None]
        ind_ref[0, t : t + 1, :] = idx[None]
        work = jnp.where(iota_m == idx[None, :], -1.0, work)
    v_ref[0, K:, :] = jnp.zeros((KPAD - K, C), jnp.float32)
    ind_ref[0, K:, :] = jnp.zeros((KPAD - K, C), jnp.int32)


def _softmax_topk(memoryMartix):
    return pl.pallas_call(
        _softmax_topk_body,
        grid=(A,),
        in_specs=[pl.BlockSpec((1, M, C), lambda a: (a, 0, 0))],
        out_specs=[
            pl.BlockSpec((1, M, C), lambda a: (a, 0, 0)),
            pl.BlockSpec((1, KPAD, C), lambda a: (a, 0, 0)),
            pl.BlockSpec((1, KPAD, C), lambda a: (a, 0, 0)),
        ],
        out_shape=[
            jax.ShapeDtypeStruct((A, M, C), jnp.float32),
            jax.ShapeDtypeStruct((A, KPAD, C), jnp.int32),
            jax.ShapeDtypeStruct((A, KPAD, C), jnp.float32),
        ],
    )(memoryMartix)


def _gather_fuse(sm, ind, v, src1):
    mesh = plsc.VectorSubcoreMesh(core_axis_name="c", subcore_axis_name="s")

    @functools.partial(
        pl.kernel,
        out_type=jax.ShapeDtypeStruct((A, A, C, C), jnp.float32),
        mesh=mesh,
        scratch_types=[
            pltpu.VMEM((M, C2W), jnp.float32),
            pltpu.VMEM((IPG, KPAD, C), jnp.int32),
            pltpu.VMEM((IPG, KPAD, C), jnp.float32),
            pltpu.VMEM((C, C2W), jnp.float32),
            pltpu.VMEM((C, C2W), jnp.float32),
            pltpu.SemaphoreType.DMA,
        ],
    )
    def sc_kernel(sm_hbm, ind_hbm, v_hbm, src1_hbm, out_hbm,
                  chunk, ind_b, v_b, s_b, o_b, sem):
        del sem
        cid = lax.axis_index("c")
        sid = lax.axis_index("s")
        wid = sid * 2 + cid
        c2g = wid % NC2CHUNK
        ig = wid // NC2CHUNK
        c2lo = c2g * C2W

        pltpu.sync_copy(ind_hbm.at[pl.ds(ig * IPG, IPG)], ind_b)
        pltpu.sync_copy(v_hbm.at[pl.ds(ig * IPG, IPG)], v_b)
        iota = lax.iota(jnp.int32, 16)

        def a_body(a, carry0):
            pltpu.sync_copy(sm_hbm.at[a, :, pl.ds(c2lo, C2W)], chunk)

            def i_body(ii, carry1):
                i = ig * IPG + ii
                pltpu.sync_copy(src1_hbm.at[a, i, :, pl.ds(c2lo, C2W)], s_b)
                for cb in range(C // 16):
                    rows = iota + cb * 16
                    accs = tuple(
                        plsc.load_gather(
                            s_b, [rows, jnp.full((16,), c2, jnp.int32)])
                        for c2 in range(C2W)
                    )

                    def t_body(t, accs):
                        ind_vec = ind_b[ii, t, pl.ds(cb * 16, 16)]
                        v_vec = v_b[ii, t, pl.ds(cb * 16, 16)]
                        out = []
                        for c2 in range(C2W):
                            g = plsc.load_gather(
                                chunk,
                                [ind_vec, jnp.full((16,), c2, jnp.int32)])
                            out.append(accs[c2] + v_vec * g)
                        return tuple(out)

                    accs = lax.fori_loop(0, K, t_body, accs)
                    for c2 in range(C2W):
                        plsc.store_scatter(
                            o_b, [rows, jnp.full((16,), c2, jnp.int32)],
                            accs[c2])
                pltpu.sync_copy(o_b, out_hbm.at[a, i, :, pl.ds(c2lo, C2W)])
                return carry1

            return lax.fori_loop(0, IPG, i_body, carry0)

        lax.fori_loop(0, A, a_body, 0)

    return sc_kernel(sm, ind, v, src1)


def kernel(src1, memoryMartix):
    sm, ind, v = _softmax_topk(memoryMartix)
    return _gather_fuse(sm, ind, v, src1)


# speedup vs baseline: 3.3861x; 3.3861x over previous
import functools

import jax
import jax.numpy as jnp
from jax import lax
from jax.experimental import pallas as pl
from jax.experimental.pallas import tpu as pltpu
from jax.experimental.pallas import tpu_sc as plsc

A = 64
M = 4096
C = 64
K = 20
KPAD = 24

NWORK = 32
NC2CHUNK = 4
C2W = C // NC2CHUNK
NIGRP = NWORK // NC2CHUNK
IPG = A // NIGRP

IB = 8


def _softmax_topk_body(mm_ref, smt_ref, ind_ref, v_ref):
    x = mm_ref[0]
    xm = jnp.max(x, axis=0, keepdims=True)
    e = jnp.exp(x - xm)
    s = jnp.sum(e, axis=0, keepdims=True)
    sm = e / s
    smt_ref[0] = sm.T
    iota_m = lax.broadcasted_iota(jnp.int32, (M, C), 0)
    work = sm
    for t in range(K):
        mx = jnp.max(work, axis=0)
        cand = jnp.where(work == mx[None, :], iota_m, M)
        idx = jnp.min(cand, axis=0)
        v_ref[0, t : t + 1, :] = mx[None]
        ind_ref[0, t : t + 1, :] = idx[None]
        work = jnp.where(iota_m == idx[None, :], -1.0, work)
    v_ref[0, K:, :] = jnp.zeros((KPAD - K, C), jnp.float32)
    ind_ref[0, K:, :] = jnp.zeros((KPAD - K, C), jnp.int32)


def _softmax_topk(memoryMartix):
    return pl.pallas_call(
        _softmax_topk_body,
        grid=(A,),
        in_specs=[pl.BlockSpec((1, M, C), lambda a: (a, 0, 0))],
        out_specs=[
            pl.BlockSpec((1, C, M), lambda a: (a, 0, 0)),
            pl.BlockSpec((1, KPAD, C), lambda a: (a, 0, 0)),
            pl.BlockSpec((1, KPAD, C), lambda a: (a, 0, 0)),
        ],
        out_shape=[
            jax.ShapeDtypeStruct((A, C, M), jnp.float32),
            jax.ShapeDtypeStruct((A, KPAD, C), jnp.int32),
            jax.ShapeDtypeStruct((A, KPAD, C), jnp.float32),
        ],
    )(memoryMartix)


def _gather_stage(smt, ind, v):
    mesh = plsc.VectorSubcoreMesh(core_axis_name="c", subcore_axis_name="s")

    @functools.partial(
        pl.kernel,
        out_type=jax.ShapeDtypeStruct((NC2CHUNK, A, A, C, C2W), jnp.float32),
        mesh=mesh,
        compiler_params=pltpu.CompilerParams(
            use_tc_tiling_on_sc=False, needs_layout_passes=False),
        scratch_types=[
            pltpu.VMEM((C2W, M), jnp.float32),
            pltpu.VMEM((IPG, KPAD, C), jnp.int32),
            pltpu.VMEM((IPG, KPAD, C), jnp.float32),
            pltpu.VMEM((C, C2W), jnp.float32),
        ],
    )
    def sc_kernel(smt_hbm, ind_hbm, v_hbm, g_hbm, chunk, ind_b, v_b, o_b):
        cid = lax.axis_index("c")
        sid = lax.axis_index("s")
        wid = sid * 2 + cid
        c2g = wid % NC2CHUNK
        ig = wid // NC2CHUNK
        c2lo = c2g * C2W

        pltpu.sync_copy(ind_hbm.at[pl.ds(ig * IPG, IPG)], ind_b)
        pltpu.sync_copy(v_hbm.at[pl.ds(ig * IPG, IPG)], v_b)
        iota = lax.iota(jnp.int32, 16)
        zero = jnp.zeros((16,), jnp.float32)

        def a_body(a, carry0):
            pltpu.sync_copy(smt_hbm.at[a, pl.ds(c2lo, C2W)], chunk)

            def i_body(ii, carry1):
                i = ig * IPG + ii
                for cb in range(C // 16):
                    rows = iota + cb * 16
                    accs = (zero,) * C2W

                    def t_body(t, accs):
                        ind_vec = ind_b[ii, t, pl.ds(cb * 16, 16)]
                        v_vec = v_b[ii, t, pl.ds(cb * 16, 16)]
                        out = []
                        for c2 in range(C2W):
                            g = plsc.load_gather(
                                chunk,
                                [jnp.full((16,), c2, jnp.int32), ind_vec])
                            out.append(accs[c2] + v_vec * g)
                        return tuple(out)

                    accs = lax.fori_loop(0, K, t_body, accs)
                    for c2 in range(C2W):
                        plsc.store_scatter(
                            o_b, [rows, jnp.full((16,), c2, jnp.int32)],
                            accs[c2])
                pltpu.sync_copy(o_b, g_hbm.at[c2g, a, i])
                return carry1

            return lax.fori_loop(0, IPG, i_body, carry0)

        lax.fori_loop(0, A, a_body, 0)

    return sc_kernel(smt, ind, v)


def _add_body(src_ref, g_ref, out_ref):
    g = g_ref[...]
    parts = [g[k, 0] for k in range(NC2CHUNK)]
    out_ref[0] = src_ref[0] + jnp.concatenate(parts, axis=-1)


def _add_stage(src1, gterm):
    return pl.pallas_call(
        _add_body,
        grid=(A, A // IB),
        in_specs=[
            pl.BlockSpec((1, IB, C, C), lambda a, ib: (a, ib, 0, 0)),
            pl.BlockSpec((NC2CHUNK, 1, IB, C, C2W),
                         lambda a, ib: (0, a, ib, 0, 0)),
        ],
        out_specs=pl.BlockSpec((1, IB, C, C), lambda a, ib: (a, ib, 0, 0)),
        out_shape=jax.ShapeDtypeStruct((A, A, C, C), jnp.float32),
    )(src1, gterm)


def kernel(src1, memoryMartix):
    smt, ind, v = _softmax_topk(memoryMartix)
    gterm = _gather_stage(smt, ind, v)
    return _add_stage(src1, gterm)


# speedup vs baseline: 4.1407x; 1.2228x over previous
import functools

import jax
import jax.numpy as jnp
from jax import lax
from jax.experimental import pallas as pl
from jax.experimental.pallas import tpu as pltpu
from jax.experimental.pallas import tpu_sc as plsc

A = 64
M = 4096
C = 64
K = 20
KPAD = 24

NWORK = 32
NC2CHUNK = 4
C2W = C // NC2CHUNK
NIGRP = NWORK // NC2CHUNK
IPG = A // NIGRP

IB = 8


def _softmax_topk_body(mm_ref, smt_ref, ind_ref, v_ref):
    x = mm_ref[0]
    xm = jnp.max(x, axis=0, keepdims=True)
    e = jnp.exp(x - xm)
    s = jnp.sum(e, axis=0, keepdims=True)
    sm = e / s
    smt_ref[0] = sm.T
    iota_m = lax.broadcasted_iota(jnp.int32, (M, C), 0)
    bits = lax.bitcast_convert_type(sm, jnp.int32)
    work = (bits & jnp.int32(~(M - 1))) | (jnp.int32(M - 1) - iota_m)
    neg_inf = jnp.int32(-(2**31))
    for t in range(K):
        mx = jnp.max(work, axis=0)
        work = jnp.where(work == mx[None, :], neg_inf, work)
        idx = jnp.int32(M - 1) - (mx & jnp.int32(M - 1))
        val = lax.bitcast_convert_type(mx & jnp.int32(~(M - 1)), jnp.float32)
        v_ref[0, t : t + 1, :] = val[None]
        ind_ref[0, t : t + 1, :] = idx[None]
    v_ref[0, K:, :] = jnp.zeros((KPAD - K, C), jnp.float32)
    ind_ref[0, K:, :] = jnp.zeros((KPAD - K, C), jnp.int32)


def _softmax_topk(memoryMartix):
    return pl.pallas_call(
        _softmax_topk_body,
        grid=(A,),
        in_specs=[pl.BlockSpec((1, M, C), lambda a: (a, 0, 0))],
        out_specs=[
            pl.BlockSpec((1, C, M), lambda a: (a, 0, 0)),
            pl.BlockSpec((1, KPAD, C), lambda a: (a, 0, 0)),
            pl.BlockSpec((1, KPAD, C), lambda a: (a, 0, 0)),
        ],
        out_shape=[
            jax.ShapeDtypeStruct((A, C, M), jnp.float32),
            jax.ShapeDtypeStruct((A, KPAD, C), jnp.int32),
            jax.ShapeDtypeStruct((A, KPAD, C), jnp.float32),
        ],
    )(memoryMartix)


def _gather_stage(smt, ind, v):
    mesh = plsc.VectorSubcoreMesh(core_axis_name="c", subcore_axis_name="s")

    @functools.partial(
        pl.kernel,
        out_type=jax.ShapeDtypeStruct((NC2CHUNK, A, A, C, C2W), jnp.float32),
        mesh=mesh,
        compiler_params=pltpu.CompilerParams(
            use_tc_tiling_on_sc=False, needs_layout_passes=False),
        scratch_types=[
            pltpu.VMEM((C2W, M), jnp.float32),
            pltpu.VMEM((IPG, KPAD, C), jnp.int32),
            pltpu.VMEM((IPG, KPAD, C), jnp.float32),
            pltpu.VMEM((C, C2W), jnp.float32),
        ],
    )
    def sc_kernel(smt_hbm, ind_hbm, v_hbm, g_hbm, chunk, ind_b, v_b, o_b):
        cid = lax.axis_index("c")
        sid = lax.axis_index("s")
        wid = sid * 2 + cid
        c2g = wid % NC2CHUNK
        ig = wid // NC2CHUNK
        c2lo = c2g * C2W

        pltpu.sync_copy(ind_hbm.at[pl.ds(ig * IPG, IPG)], ind_b)
        pltpu.sync_copy(v_hbm.at[pl.ds(ig * IPG, IPG)], v_b)
        iota = lax.iota(jnp.int32, 16)
        zero = jnp.zeros((16,), jnp.float32)

        def a_body(a, carry0):
            pltpu.sync_copy(smt_hbm.at[a, pl.ds(c2lo, C2W)], chunk)

            def i_body(ii, carry1):
                i = ig * IPG + ii
                for cb in range(C // 16):
                    rows = iota + cb * 16
                    accs = (zero,) * C2W

                    def t_body(t, accs):
                        ind_vec = ind_b[ii, t, pl.ds(cb * 16, 16)]
                        v_vec = v_b[ii, t, pl.ds(cb * 16, 16)]
                        out = []
                        for c2 in range(C2W):
                            g = plsc.load_gather(
                                chunk,
                                [jnp.full((16,), c2, jnp.int32), ind_vec])
                            out.append(accs[c2] + v_vec * g)
                        return tuple(out)

                    accs = lax.fori_loop(0, K, t_body, accs)
                    for c2 in range(C2W):
                        plsc.store_scatter(
                            o_b, [rows, jnp.full((16,), c2, jnp.int32)],
                            accs[c2])
                pltpu.sync_copy(o_b, g_hbm.at[c2g, a, i])
                return carry1

            return lax.fori_loop(0, IPG, i_body, carry0)

        lax.fori_loop(0, A, a_body, 0)

    return sc_kernel(smt, ind, v)


def _add_body(src_ref, g_ref, out_ref):
    g = g_ref[...]
    parts = [g[k, 0] for k in range(NC2CHUNK)]
    out_ref[0] = src_ref[0] + jnp.concatenate(parts, axis=-1)


def _add_stage(src1, gterm):
    return pl.pallas_call(
        _add_body,
        grid=(A, A // IB),
        in_specs=[
            pl.BlockSpec((1, IB, C, C), lambda a, ib: (a, ib, 0, 0)),
            pl.BlockSpec((NC2CHUNK, 1, IB, C, C2W),
                         lambda a, ib: (0, a, ib, 0, 0)),
        ],
        out_specs=pl.BlockSpec((1, IB, C, C), lambda a, ib: (a, ib, 0, 0)),
        out_shape=jax.ShapeDtypeStruct((A, A, C, C), jnp.float32),
    )(src1, gterm)


def kernel(src1, memoryMartix):
    smt, ind, v = _softmax_topk(memoryMartix)
    gterm = _gather_stage(smt, ind, v)
    return _add_stage(src1, gterm)
